# baseline (speedup 1.0000x reference)
import functools

import jax
import jax.numpy as jnp
import numpy as np
from jax import lax
from jax.experimental import pallas as pl
from jax.experimental.pallas import tpu as pltpu
from jax.experimental.pallas import tpu_sc as plsc

N_LEVELS = 16
LEVEL_DIM = 2
BASE_RES = 16
DESIRED_RES = 2048
LOG2_T = 19
T = 1 << LOG2_T
DIVIDE = 1.5
RADIUS = 4.0 * 0.05
K_NN = 20
PRIME1 = np.uint32(2654435761)
PRIME2 = np.uint32(805459861)

R2 = np.float32(RADIUS * RADIUS)
P1_I32 = np.int32(np.uint32(PRIME1).view(np.int32))
P2_I32 = np.int32(np.uint32(PRIME2).view(np.int32))
_SCALE = 2.0 ** (np.log2(DESIRED_RES / BASE_RES) / (N_LEVELS - 1))
RES_LEVELS = [float(int(np.floor(BASE_RES * _SCALE ** l))) for l in range(N_LEVELS)]

NP_PART = 10000
NQ = 2048
NTILES = 32
QPT = NQ // NTILES
CAP = 128
SUP = 25
GRP = 5
NGRP = (10000 // 16) // GRP
NCHUNK = NP_PART // 16
LANES = 16
INF_F32 = np.float32(np.inf)
IMAX = np.int32(2**30)


def _sc_body(px_h, py_h, pz_h, qx_h, qy_h, qz_h, tab,
             smx_o, smy_o, smz_o, dens_o, feat,
             px, py, pz, qx, qy, qz,
             cd2, cidx, csel,
             smxv, smyv, smzv, densv,
             idxb, gbuf, featb, gmin, cnt_ref, sem):
    wid = lax.axis_index("s") * 2 + lax.axis_index("c")
    base = wid * QPT

    pltpu.sync_copy(px_h, px)
    pltpu.sync_copy(py_h, py)
    pltpu.sync_copy(pz_h, pz)
    pltpu.sync_copy(qx_h.at[pl.ds(base, QPT)], qx.at[pl.ds(0, QPT)])
    pltpu.sync_copy(qy_h.at[pl.ds(base, QPT)], qy.at[pl.ds(0, QPT)])
    pltpu.sync_copy(qz_h.at[pl.ds(base, QPT)], qz.at[pl.ds(0, QPT)])

    lane = lax.iota(jnp.int32, LANES)
    infv = jnp.full((LANES,), INF_F32, jnp.float32)

    def q_body(qi, _):
        qiv = jnp.full((LANES,), qi, jnp.int32)
        qxv = plsc.load_gather(qx, [qiv])
        qyv = plsc.load_gather(qy, [qiv])
        qzv = plsc.load_gather(qz, [qiv])

        for j in range(CAP // LANES):
            cd2[pl.ds(j * LANES, LANES)] = infv

        cnt_ref[0] = 0

        def d2_at(o):
            dx = qxv - px[pl.ds(o, LANES)]
            dy = qyv - py[pl.ds(o, LANES)]
            dz = qzv - pz[pl.ds(o, LANES)]
            return (dx * dx + dy * dy) + dz * dz

        @plsc.parallel_loop(0, NGRP, step=1, unroll=4)
        def _p1(gi):
            o0 = gi * (LANES * GRP)
            mng = None
            for s in range(GRP):
                o = pl.multiple_of(o0 + s * LANES, LANES)
                d2 = d2_at(o)
                mng = d2 if mng is None else jnp.minimum(mng, d2)
            gmin[pl.ds(gi * LANES, LANES)] = mng

        def p2(js, _carry):
            o0 = js * (LANES * SUP)
            vs = [gmin[pl.ds((js * (SUP // GRP) + g) * LANES, LANES)]
                  for g in range(SUP // GRP)]
            mn = vs[0]
            for g in range(1, SUP // GRP):
                mn = jnp.minimum(mn, vs[g])

            @pl.when(jnp.min(mn) <= R2)
            def _():
                for g in range(SUP // GRP):
                    @pl.when(jnp.min(vs[g]) <= R2)
                    def _(g=g):
                        cv = jnp.full((LANES,), cnt_ref[0], jnp.int32)
                        for s in range(GRP):
                            o = pl.multiple_of(o0 + (g * GRP + s) * LANES, LANES)
                            d2 = d2_at(o)
                            m = d2 <= R2
                            ones = jnp.where(m, 1, 0).astype(jnp.int32)
                            pos = cv + plsc.cumsum(ones) - 1
                            pos = jnp.minimum(pos, CAP - 1)
                            plsc.store_scatter(cd2, [pos], d2, mask=m)
                            plsc.store_scatter(cidx, [pos], lane + (o0 + (g * GRP + s) * LANES), mask=m)
                            cv = cv + plsc.all_reduce_population_count(m)
                        cnt_ref[0] = jnp.max(cv)

            return 0

        lax.fori_loop(0, NCHUNK // SUP, p2, 0)
        cnt_s = cnt_ref[0]

        def big_case(_):
            for j in range(CAP // LANES):
                csel[pl.ds(j * LANES, LANES)] = cd2[pl.ds(j * LANES, LANES)]

            def w_cond(st):
                return st[0] < K_NN

            def w_body(st):
                removed = st[0]
                m0 = csel[pl.ds(0, LANES)]
                for j in range(1, CAP // LANES):
                    m0 = jnp.minimum(m0, csel[pl.ds(j * LANES, LANES)])
                mv = jnp.min(m0)
                mvv = jnp.full((LANES,), mv, jnp.float32)
                ndup = jnp.zeros((LANES,), jnp.int32)
                for j in range(CAP // LANES):
                    cj = csel[pl.ds(j * LANES, LANES)]
                    eq = cj == mvv
                    ndup = ndup + plsc.all_reduce_population_count(eq)
                    csel[pl.ds(j * LANES, LANES)] = jnp.where(eq, infv, cj)
                return removed + jnp.max(ndup), mv

            _, tau = lax.while_loop(w_cond, w_body, (jnp.int32(0), jnp.float32(0)))
            tauv = jnp.full((LANES,), tau, jnp.float32)

            nlt = jnp.zeros((LANES,), jnp.int32)
            for j in range(CAP // LANES):
                cj = cd2[pl.ds(j * LANES, LANES)]
                nlt = nlt + plsc.all_reduce_population_count(cj < tauv)
            need = jnp.int32(K_NN) - jnp.max(nlt)

            def i_cond(st):
                return st[0] < need

            def i_body(st):
                prev = st[1]
                prevv = jnp.full((LANES,), prev, jnp.int32)
                imin = jnp.full((LANES,), IMAX, jnp.int32)
                for j in range(CAP // LANES):
                    cj = cd2[pl.ds(j * LANES, LANES)]
                    ij = cidx[pl.ds(j * LANES, LANES)]
                    cand = jnp.where((cj == tauv) & (ij > prevv), ij, IMAX)
                    imin = jnp.minimum(imin, cand)
                return st[0] + 1, jnp.min(imin)

            _, icut = lax.while_loop(i_cond, i_body, (jnp.int32(0), jnp.int32(-1)))
            return tau, icut

        def small_case(_):
            return jnp.float32(R2), IMAX

        tau, icut = lax.cond(cnt_s > K_NN, big_case, small_case, None)
        tauv = jnp.full((LANES,), tau, jnp.float32)
        icutv = jnp.full((LANES,), icut, jnp.int32)

        ws = jnp.zeros((LANES,), jnp.float32)
        sx = jnp.zeros((LANES,), jnp.float32)
        sy = jnp.zeros((LANES,), jnp.float32)
        sz = jnp.zeros((LANES,), jnp.float32)
        nn = jnp.zeros((LANES,), jnp.int32)
        for j in range(CAP // LANES):
            d2c = cd2[pl.ds(j * LANES, LANES)]
            pic = cidx[pl.ds(j * LANES, LANES)]
            sel = (d2c < tauv) | ((d2c == tauv) & (pic <= icutv))
            live = sel & (d2c != 0.0)
            w = jnp.maximum(R2 - d2c, 0.0)
            wm = jnp.where(live, (w * w) * w, 0.0)
            ws = ws + wm
            nn = nn + jnp.where(live, 1, 0).astype(jnp.int32)
            gx = plsc.load_gather(px, [pic], mask=sel)
            gy = plsc.load_gather(py, [pic], mask=sel)
            gz = plsc.load_gather(pz, [pic], mask=sel)
            sx = sx + wm * gx
            sy = sy + wm * gy
            sz = sz + wm * gz

        wsumv = jnp.full((LANES,), jnp.sum(ws), jnp.float32)
        hasv = jnp.full((LANES,), jnp.sum(nn), jnp.int32) > 0
        denv = jnp.maximum(wsumv, jnp.float32(1e-12))
        resx = jnp.where(hasv, jnp.full((LANES,), jnp.sum(sx), jnp.float32) / denv, qxv)
        resy = jnp.where(hasv, jnp.full((LANES,), jnp.sum(sy), jnp.float32) / denv, qyv)
        resz = jnp.where(hasv, jnp.full((LANES,), jnp.sum(sz), jnp.float32) / denv, qzv)
        lane0 = lane == 0
        plsc.store_scatter(smxv, [qiv], resx, mask=lane0)
        plsc.store_scatter(smyv, [qiv], resy, mask=lane0)
        plsc.store_scatter(smzv, [qiv], resz, mask=lane0)
        plsc.store_scatter(densv, [qiv], wsumv, mask=lane0)
        return 0

    lax.fori_loop(0, QPT, q_body, 0)

    pltpu.sync_copy(smxv.at[pl.ds(0, QPT)], smx_o.at[pl.ds(base, QPT)])
    pltpu.sync_copy(smyv.at[pl.ds(0, QPT)], smy_o.at[pl.ds(base, QPT)])
    pltpu.sync_copy(smzv.at[pl.ds(0, QPT)], smz_o.at[pl.ds(base, QPT)])
    pltpu.sync_copy(densv.at[pl.ds(0, QPT)], dens_o.at[pl.ds(base, QPT)])

    rowvs = [lane + c * LANES for c in range(8)]

    def h_body(ch, _):
        o = pl.multiple_of(ch * LANES, LANES)
        xin = qx[pl.ds(o, LANES)] / DIVIDE
        yin = qy[pl.ds(o, LANES)] / DIVIDE
        zin = qz[pl.ds(o, LANES)] / DIVIDE
        xnx = jnp.clip((xin + 1.0) * 0.5, 0.0, 1.0 - 1e-6)
        xny = jnp.clip((yin + 1.0) * 0.5, 0.0, 1.0 - 1e-6)
        xnz = jnp.clip((zin + 1.0) * 0.5, 0.0, 1.0 - 1e-6)

        def corners(l):
            res = np.float32(RES_LEVELS[l])
            ix = (xnx * res).astype(jnp.int32)
            iy = (xny * res).astype(jnp.int32)
            iz = (xnz * res).astype(jnp.int32)
            return ix, iy, iz

        for l in range(N_LEVELS):
            ix, iy, iz = corners(l)
            a = [ix, ix + 1]
            b = [iy * P1_I32, (iy + 1) * P1_I32]
            c3 = [iz * P2_I32, (iz + 1) * P2_I32]
            for c in range(8):
                h = (a[c & 1] ^ b[(c >> 1) & 1] ^ c3[(c >> 2) & 1]) & jnp.int32(T - 1)
                g2 = ((h >> 7) << 8) + (h & 127) + jnp.int32(l * 2 * T)
                idxb[pl.ds(l * 256 + c * LANES, LANES)] = g2
                idxb[pl.ds(l * 256 + 128 + c * LANES, LANES)] = g2 + 128

        descs = []
        for l in range(N_LEVELS):
            descs.append(pltpu.async_copy(
                tab.at[idxb.at[pl.ds(l * 256, 128)]],
                gbuf.at[pl.ds(l * 256, 128)], sem))
            descs.append(pltpu.async_copy(
                tab.at[idxb.at[pl.ds(l * 256 + 128, 128)]],
                gbuf.at[pl.ds(l * 256 + 128, 128)], sem))
        for d in descs:
            d.wait()

        for l in range(N_LEVELS):
            res = np.float32(RES_LEVELS[l])
            posx = xnx * res
            posy = xny * res
            posz = xnz * res
            fx = posx - posx.astype(jnp.int32).astype(jnp.float32)
            fy = posy - posy.astype(jnp.int32).astype(jnp.float32)
            fz = posz - posz.astype(jnp.int32).astype(jnp.float32)
            wxs = [1.0 - fx, fx]
            wys = [1.0 - fy, fy]
            wzs = [1.0 - fz, fz]
            f0 = jnp.zeros((LANES,), jnp.float32)
            f1 = jnp.zeros((LANES,), jnp.float32)
            for c in range(8):
                wt = wxs[c & 1] * wys[(c >> 1) & 1] * wzs[(c >> 2) & 1]
                rowv = rowvs[c] + l * 256
                g0 = plsc.load_gather(gbuf, [rowv])
                g1 = plsc.load_gather(gbuf, [rowv + 128])
                f0 = f0 + wt * g0
                f1 = f1 + wt * g1
            featb[pl.ds((2 * l) * QPT + o, LANES)] = f0
            featb[pl.ds((2 * l + 1) * QPT + o, LANES)] = f1
        return 0

    lax.fori_loop(0, QPT // LANES, h_body, 0)
    for r in range(2 * N_LEVELS):
        pltpu.sync_copy(featb.at[pl.ds(r * QPT, QPT)],
                        feat.at[pl.ds(r * NQ + base, QPT)])


def _softplus100(x):
    z = 100.0 * x
    return (jnp.maximum(z, 0.0) + jnp.log1p(jnp.exp(-jnp.abs(z)))) / 100.0


def _mlp_body(inp, sm, dn, ft, W0, b0, W1, b1, W2, b2, W3, b3, W4, b4, out):
    hit = inp[...]
    smo = sm[...]
    dnv = dn[...]
    rows = [hit]
    for i in range(10):
        f = float(2.0 ** i)
        rows.append(jnp.sin(f * hit))
        rows.append(jnp.cos(f * hit))
    rows.append(smo)
    for i in range(10):
        f = float(2.0 ** i)
        rows.append(jnp.sin(f * smo))
        rows.append(jnp.cos(f * smo))
    rows.append(dnv)
    for i in range(4):
        f = float(2.0 ** i)
        rows.append(jnp.sin(f * dnv))
        rows.append(jnp.cos(f * dnv))
    rows.append(ft[...])
    x = jnp.concatenate(rows, axis=0)
    Ws = [W0, W1, W2, W3, W4]
    bs = [b0, b1, b2, b3, b4]
    for i in range(5):
        x = jnp.dot(Ws[i][...], x, preferred_element_type=jnp.float32)
        x = x + bs[i][...][:, None]
        if i < 4:
            x = _softplus100(x)
    out[...] = x


def kernel(input, physical_particles, tables, W0, b0, W1, b1, W2, b2, W3, b3, W4, b4):
    B, N, _ = input.shape
    inp2 = input.reshape(B * N, 3)
    qT = inp2.T
    pT = physical_particles.T
    tab = tables.reshape(N_LEVELS, T // 128, 128, LEVEL_DIM)
    tab = tab.transpose(0, 1, 3, 2).reshape(N_LEVELS * T * LEVEL_DIM)
    px_h, py_h, pz_h = pT[0], pT[1], pT[2]
    qx_h, qy_h, qz_h = qT[0], qT[1], qT[2]

    sc = pl.kernel(
        _sc_body,
        out_type=[
            jax.ShapeDtypeStruct((NQ,), jnp.float32),
            jax.ShapeDtypeStruct((NQ,), jnp.float32),
            jax.ShapeDtypeStruct((NQ,), jnp.float32),
            jax.ShapeDtypeStruct((NQ,), jnp.float32),
            jax.ShapeDtypeStruct((2 * N_LEVELS * NQ,), jnp.float32),
        ],
        mesh=plsc.VectorSubcoreMesh(
            core_axis_name="c", subcore_axis_name="s", num_cores=2, num_subcores=16
        ),
        compiler_params=pltpu.CompilerParams(needs_layout_passes=False, use_tc_tiling_on_sc=False),
        scratch_types=[
            pltpu.VMEM((NP_PART,), jnp.float32),
            pltpu.VMEM((NP_PART,), jnp.float32),
            pltpu.VMEM((NP_PART,), jnp.float32),
            pltpu.VMEM((128,), jnp.float32),
            pltpu.VMEM((128,), jnp.float32),
            pltpu.VMEM((128,), jnp.float32),
            pltpu.VMEM((CAP,), jnp.float32),
            pltpu.VMEM((CAP,), jnp.int32),
            pltpu.VMEM((CAP,), jnp.float32),
            pltpu.VMEM((128,), jnp.float32),
            pltpu.VMEM((128,), jnp.float32),
            pltpu.VMEM((128,), jnp.float32),
            pltpu.VMEM((128,), jnp.float32),
            pltpu.VMEM((N_LEVELS * 256,), jnp.int32),
            pltpu.VMEM((N_LEVELS * 256,), jnp.float32),
            pltpu.VMEM((2 * N_LEVELS * QPT,), jnp.float32),
            pltpu.VMEM((NGRP * LANES,), jnp.float32),
            pltpu.SMEM((8,), jnp.int32),
            pltpu.SemaphoreType.DMA,
        ],
    )
    smx, smy, smz, densf, featf = sc(px_h, py_h, pz_h, qx_h, qy_h, qz_h, tab)
    smT = jnp.stack([smx, smy, smz])
    dens = densf.reshape(1, NQ)
    feat = featf.reshape(2 * N_LEVELS, NQ)

    NB = 512
    dims_in = [167, 256, 256, 256, 256]
    dims_out = [256, 256, 256, 256, 257]
    w_specs = []
    for i in range(5):
        w_specs.append(pl.BlockSpec((dims_out[i], dims_in[i]), lambda n: (0, 0)))
        w_specs.append(pl.BlockSpec((dims_out[i],), lambda n: (0,)))
    out = pl.pallas_call(
        _mlp_body,
        grid=(NQ // NB,),
        in_specs=[
            pl.BlockSpec((3, NB), lambda n: (0, n)),
            pl.BlockSpec((3, NB), lambda n: (0, n)),
            pl.BlockSpec((1, NB), lambda n: (0, n)),
            pl.BlockSpec((2 * N_LEVELS, NB), lambda n: (0, n)),
        ] + w_specs,
        out_specs=pl.BlockSpec((257, NB), lambda n: (0, n)),
        out_shape=jax.ShapeDtypeStruct((257, NQ), jnp.float32),
    )(qT, smT, dens, feat, W0, b0, W1, b1, W2, b2, W3, b3, W4, b4)
    return out.T


# speedup vs baseline: 1.3033x; 1.3033x over previous
import jax
import jax.numpy as jnp
import numpy as np
from jax import lax
from jax.experimental import pallas as pl
from jax.experimental.pallas import tpu as pltpu
from jax.experimental.pallas import tpu_sc as plsc

N_LEVELS = 16
LEVEL_DIM = 2
BASE_RES = 16
DESIRED_RES = 2048
LOG2_T = 19
T = 1 << LOG2_T
DIVIDE = 1.5
RADIUS = 4.0 * 0.05
K_NN = 20
PRIME1 = np.uint32(2654435761)
PRIME2 = np.uint32(805459861)

R2 = np.float32(RADIUS * RADIUS)
P1_I32 = np.int32(np.uint32(PRIME1).view(np.int32))
P2_I32 = np.int32(np.uint32(PRIME2).view(np.int32))
_SCALE = 2.0 ** (np.log2(DESIRED_RES / BASE_RES) / (N_LEVELS - 1))
RES_LEVELS = [float(int(np.floor(BASE_RES * _SCALE ** l))) for l in range(N_LEVELS)]

NP_PART = 10000
NQ = 2048
NTILES = 32
QPT = NQ // NTILES
CAP = 128
SUP = 25
NCHUNK = NP_PART // 16
LANES = 16
INF_F32 = np.float32(np.inf)
IMAX = np.int32(2**30)


def _sc_body(px_h, py_h, pz_h, qx_h, qy_h, qz_h, tab,
             smx_o, smy_o, smz_o, dens_o, feat,
             px, py, pz, qx, qy, qz,
             cd2, cidx, csel,
             smxv, smyv, smzv, densv,
             idxb, gbuf, featb, cnt_ref, sem):
    wid = lax.axis_index("s") * 2 + lax.axis_index("c")
    base = wid * QPT

    pltpu.sync_copy(px_h, px)
    pltpu.sync_copy(py_h, py)
    pltpu.sync_copy(pz_h, pz)
    pltpu.sync_copy(qx_h.at[pl.ds(base, QPT)], qx.at[pl.ds(0, QPT)])
    pltpu.sync_copy(qy_h.at[pl.ds(base, QPT)], qy.at[pl.ds(0, QPT)])
    pltpu.sync_copy(qz_h.at[pl.ds(base, QPT)], qz.at[pl.ds(0, QPT)])

    lane = lax.iota(jnp.int32, LANES)
    infv = jnp.full((LANES,), INF_F32, jnp.float32)

    def q_body(qi, _):
        qiv = jnp.full((LANES,), qi, jnp.int32)
        qxv = plsc.load_gather(qx, [qiv])
        qyv = plsc.load_gather(qy, [qiv])
        qzv = plsc.load_gather(qz, [qiv])

        for j in range(CAP // LANES):
            cd2[pl.ds(j * LANES, LANES)] = infv

        cnt_ref[0] = 0

        def chunk(js, _carry):
            o0 = js * (LANES * SUP)
            d2s = []
            for s in range(SUP):
                o = pl.multiple_of(o0 + s * LANES, LANES)
                dx = qxv - px[pl.ds(o, LANES)]
                dy = qyv - py[pl.ds(o, LANES)]
                dz = qzv - pz[pl.ds(o, LANES)]
                d2s.append((dx * dx + dy * dy) + dz * dz)
            mn = d2s[0]
            for s in range(1, SUP):
                mn = jnp.minimum(mn, d2s[s])

            @pl.when(jnp.min(mn) <= R2)
            def _():
                cv = jnp.full((LANES,), cnt_ref[0], jnp.int32)
                for s in range(SUP):
                    m = d2s[s] <= R2
                    ones = jnp.where(m, 1, 0).astype(jnp.int32)
                    pos = cv + plsc.cumsum(ones) - 1
                    pos = jnp.minimum(pos, CAP - 1)
                    plsc.store_scatter(cd2, [pos], d2s[s], mask=m)
                    plsc.store_scatter(cidx, [pos], lane + (o0 + s * LANES), mask=m)
                    cv = cv + plsc.all_reduce_population_count(m)
                cnt_ref[0] = jnp.max(cv)

            return 0

        lax.fori_loop(0, NCHUNK // SUP, chunk, 0)
        cnt_s = cnt_ref[0]

        def big_case(_):
            for j in range(CAP // LANES):
                csel[pl.ds(j * LANES, LANES)] = cd2[pl.ds(j * LANES, LANES)]

            def w_cond(st):
                return st[0] < K_NN

            def w_body(st):
                removed = st[0]
                m0 = csel[pl.ds(0, LANES)]
                for j in range(1, CAP // LANES):
                    m0 = jnp.minimum(m0, csel[pl.ds(j * LANES, LANES)])
                mv = jnp.min(m0)
                mvv = jnp.full((LANES,), mv, jnp.float32)
                ndup = jnp.zeros((LANES,), jnp.int32)
                for j in range(CAP // LANES):
                    cj = csel[pl.ds(j * LANES, LANES)]
                    eq = cj == mvv
                    ndup = ndup + plsc.all_reduce_population_count(eq)
                    csel[pl.ds(j * LANES, LANES)] = jnp.where(eq, infv, cj)
                return removed + jnp.max(ndup), mv

            _, tau = lax.while_loop(w_cond, w_body, (jnp.int32(0), jnp.float32(0)))
            tauv = jnp.full((LANES,), tau, jnp.float32)

            nlt = jnp.zeros((LANES,), jnp.int32)
            for j in range(CAP // LANES):
                cj = cd2[pl.ds(j * LANES, LANES)]
                nlt = nlt + plsc.all_reduce_population_count(cj < tauv)
            need = jnp.int32(K_NN) - jnp.max(nlt)

            def i_cond(st):
                return st[0] < need

            def i_body(st):
                prev = st[1]
                prevv = jnp.full((LANES,), prev, jnp.int32)
                imin = jnp.full((LANES,), IMAX, jnp.int32)
                for j in range(CAP // LANES):
                    cj = cd2[pl.ds(j * LANES, LANES)]
                    ij = cidx[pl.ds(j * LANES, LANES)]
                    cand = jnp.where((cj == tauv) & (ij > prevv), ij, IMAX)
                    imin = jnp.minimum(imin, cand)
                return st[0] + 1, jnp.min(imin)

            _, icut = lax.while_loop(i_cond, i_body, (jnp.int32(0), jnp.int32(-1)))
            return tau, icut

        def small_case(_):
            return jnp.float32(R2), IMAX

        tau, icut = lax.cond(cnt_s > K_NN, big_case, small_case, None)
        tauv = jnp.full((LANES,), tau, jnp.float32)
        icutv = jnp.full((LANES,), icut, jnp.int32)

        ws = jnp.zeros((LANES,), jnp.float32)
        sx = jnp.zeros((LANES,), jnp.float32)
        sy = jnp.zeros((LANES,), jnp.float32)
        sz = jnp.zeros((LANES,), jnp.float32)
        nn = jnp.zeros((LANES,), jnp.int32)
        for j in range(CAP // LANES):
            d2c = cd2[pl.ds(j * LANES, LANES)]
            pic = cidx[pl.ds(j * LANES, LANES)]
            sel = (d2c < tauv) | ((d2c == tauv) & (pic <= icutv))
            live = sel & (d2c != 0.0)
            w = jnp.maximum(R2 - d2c, 0.0)
            wm = jnp.where(live, (w * w) * w, 0.0)
            ws = ws + wm
            nn = nn + jnp.where(live, 1, 0).astype(jnp.int32)
            gx = plsc.load_gather(px, [pic], mask=sel)
            gy = plsc.load_gather(py, [pic], mask=sel)
            gz = plsc.load_gather(pz, [pic], mask=sel)
            sx = sx + wm * gx
            sy = sy + wm * gy
            sz = sz + wm * gz

        wsumv = jnp.full((LANES,), jnp.sum(ws), jnp.float32)
        hasv = jnp.full((LANES,), jnp.sum(nn), jnp.int32) > 0
        denv = jnp.maximum(wsumv, jnp.float32(1e-12))
        resx = jnp.where(hasv, jnp.full((LANES,), jnp.sum(sx), jnp.float32) / denv, qxv)
        resy = jnp.where(hasv, jnp.full((LANES,), jnp.sum(sy), jnp.float32) / denv, qyv)
        resz = jnp.where(hasv, jnp.full((LANES,), jnp.sum(sz), jnp.float32) / denv, qzv)
        lane0 = lane == 0
        plsc.store_scatter(smxv, [qiv], resx, mask=lane0)
        plsc.store_scatter(smyv, [qiv], resy, mask=lane0)
        plsc.store_scatter(smzv, [qiv], resz, mask=lane0)
        plsc.store_scatter(densv, [qiv], wsumv, mask=lane0)
        return 0

    lax.fori_loop(0, QPT, q_body, 0)

    pltpu.sync_copy(smxv.at[pl.ds(0, QPT)], smx_o.at[pl.ds(base, QPT)])
    pltpu.sync_copy(smyv.at[pl.ds(0, QPT)], smy_o.at[pl.ds(base, QPT)])
    pltpu.sync_copy(smzv.at[pl.ds(0, QPT)], smz_o.at[pl.ds(base, QPT)])
    pltpu.sync_copy(densv.at[pl.ds(0, QPT)], dens_o.at[pl.ds(base, QPT)])

    rowvs = [lane + c * LANES for c in range(8)]

    def h_body(ch, _):
        o = pl.multiple_of(ch * LANES, LANES)
        xin = qx[pl.ds(o, LANES)] / DIVIDE
        yin = qy[pl.ds(o, LANES)] / DIVIDE
        zin = qz[pl.ds(o, LANES)] / DIVIDE
        xnx = jnp.clip((xin + 1.0) * 0.5, 0.0, 1.0 - 1e-6)
        xny = jnp.clip((yin + 1.0) * 0.5, 0.0, 1.0 - 1e-6)
        xnz = jnp.clip((zin + 1.0) * 0.5, 0.0, 1.0 - 1e-6)

        def corners(l):
            res = np.float32(RES_LEVELS[l])
            ix = (xnx * res).astype(jnp.int32)
            iy = (xny * res).astype(jnp.int32)
            iz = (xnz * res).astype(jnp.int32)
            return ix, iy, iz

        for l in range(N_LEVELS):
            ix, iy, iz = corners(l)
            a = [ix, ix + 1]
            b = [iy * P1_I32, (iy + 1) * P1_I32]
            c3 = [iz * P2_I32, (iz + 1) * P2_I32]
            for c in range(8):
                h = (a[c & 1] ^ b[(c >> 1) & 1] ^ c3[(c >> 2) & 1]) & jnp.int32(T - 1)
                g2 = ((h >> 7) << 8) + (h & 127) + jnp.int32(l * 2 * T)
                idxb[pl.ds(l * 256 + c * LANES, LANES)] = g2
                idxb[pl.ds(l * 256 + 128 + c * LANES, LANES)] = g2 + 128

        descs = []
        for l in range(N_LEVELS):
            descs.append(pltpu.async_copy(
                tab.at[idxb.at[pl.ds(l * 256, 128)]],
                gbuf.at[pl.ds(l * 256, 128)], sem))
            descs.append(pltpu.async_copy(
                tab.at[idxb.at[pl.ds(l * 256 + 128, 128)]],
                gbuf.at[pl.ds(l * 256 + 128, 128)], sem))
        for d in descs:
            d.wait()

        for l in range(N_LEVELS):
            res = np.float32(RES_LEVELS[l])
            posx = xnx * res
            posy = xny * res
            posz = xnz * res
            fx = posx - posx.astype(jnp.int32).astype(jnp.float32)
            fy = posy - posy.astype(jnp.int32).astype(jnp.float32)
            fz = posz - posz.astype(jnp.int32).astype(jnp.float32)
            wxs = [1.0 - fx, fx]
            wys = [1.0 - fy, fy]
            wzs = [1.0 - fz, fz]
            f0 = jnp.zeros((LANES,), jnp.float32)
            f1 = jnp.zeros((LANES,), jnp.float32)
            for c in range(8):
                wt = wxs[c & 1] * wys[(c >> 1) & 1] * wzs[(c >> 2) & 1]
                rowv = rowvs[c] + l * 256
                g0 = plsc.load_gather(gbuf, [rowv])
                g1 = plsc.load_gather(gbuf, [rowv + 128])
                f0 = f0 + wt * g0
                f1 = f1 + wt * g1
            featb[pl.ds((2 * l) * QPT + o, LANES)] = f0
            featb[pl.ds((2 * l + 1) * QPT + o, LANES)] = f1
        return 0

    lax.fori_loop(0, QPT // LANES, h_body, 0)
    for r in range(2 * N_LEVELS):
        pltpu.sync_copy(featb.at[pl.ds(r * QPT, QPT)],
                        feat.at[pl.ds(r * NQ + base, QPT)])


def _softplus100(x):
    z = 100.0 * x
    return (jnp.maximum(z, 0.0) + jnp.log1p(jnp.exp(-jnp.abs(z)))) / 100.0


def _mlp_body(inp, sm, dn, ft, W0, b0, W1, b1, W2, b2, W3, b3, W4, b4, out):
    hit = inp[...]
    smo = sm[...]
    dnv = dn[...]
    rows = [hit]
    for i in range(10):
        f = float(2.0 ** i)
        rows.append(jnp.sin(f * hit))
        rows.append(jnp.cos(f * hit))
    rows.append(smo)
    for i in range(10):
        f = float(2.0 ** i)
        rows.append(jnp.sin(f * smo))
        rows.append(jnp.cos(f * smo))
    rows.append(dnv)
    for i in range(4):
        f = float(2.0 ** i)
        rows.append(jnp.sin(f * dnv))
        rows.append(jnp.cos(f * dnv))
    rows.append(ft[...])
    x = jnp.concatenate(rows, axis=0)
    Ws = [W0, W1, W2, W3, W4]
    bs = [b0, b1, b2, b3, b4]
    for i in range(5):
        x = jnp.dot(Ws[i][...], x, preferred_element_type=jnp.float32)
        x = x + bs[i][...][:, None]
        if i < 4:
            x = _softplus100(x)
    out[...] = x


def kernel(input, physical_particles, tables, W0, b0, W1, b1, W2, b2, W3, b3, W4, b4):
    B, N, _ = input.shape
    inp2 = input.reshape(B * N, 3)
    qT = inp2.T
    pT = physical_particles.T
    tab = tables.reshape(N_LEVELS, T // 128, 128, LEVEL_DIM)
    tab = tab.transpose(0, 1, 3, 2).reshape(N_LEVELS * T * LEVEL_DIM)
    px_h, py_h, pz_h = pT[0], pT[1], pT[2]
    qx_h, qy_h, qz_h = qT[0], qT[1], qT[2]

    sc = pl.kernel(
        _sc_body,
        out_type=[
            jax.ShapeDtypeStruct((NQ,), jnp.float32),
            jax.ShapeDtypeStruct((NQ,), jnp.float32),
            jax.ShapeDtypeStruct((NQ,), jnp.float32),
            jax.ShapeDtypeStruct((NQ,), jnp.float32),
            jax.ShapeDtypeStruct((2 * N_LEVELS * NQ,), jnp.float32),
        ],
        mesh=plsc.VectorSubcoreMesh(
            core_axis_name="c", subcore_axis_name="s", num_cores=2, num_subcores=16
        ),
        compiler_params=pltpu.CompilerParams(needs_layout_passes=False, use_tc_tiling_on_sc=False),
        scratch_types=[
            pltpu.VMEM((NP_PART,), jnp.float32),
            pltpu.VMEM((NP_PART,), jnp.float32),
            pltpu.VMEM((NP_PART,), jnp.float32),
            pltpu.VMEM((128,), jnp.float32),
            pltpu.VMEM((128,), jnp.float32),
            pltpu.VMEM((128,), jnp.float32),
            pltpu.VMEM((CAP,), jnp.float32),
            pltpu.VMEM((CAP,), jnp.int32),
            pltpu.VMEM((CAP,), jnp.float32),
            pltpu.VMEM((128,), jnp.float32),
            pltpu.VMEM((128,), jnp.float32),
            pltpu.VMEM((128,), jnp.float32),
            pltpu.VMEM((128,), jnp.float32),
            pltpu.VMEM((N_LEVELS * 256,), jnp.int32),
            pltpu.VMEM((N_LEVELS * 256,), jnp.float32),
            pltpu.VMEM((2 * N_LEVELS * QPT,), jnp.float32),
            pltpu.SMEM((8,), jnp.int32),
            pltpu.SemaphoreType.DMA,
        ],
    )
    smx, smy, smz, densf, featf = sc(px_h, py_h, pz_h, qx_h, qy_h, qz_h, tab)
    smT = jnp.stack([smx, smy, smz])
    dens = densf.reshape(1, NQ)
    feat = featf.reshape(2 * N_LEVELS, NQ)

    NB = 512
    dims_in = [167, 256, 256, 256, 256]
    dims_out = [256, 256, 256, 256, 257]
    w_specs = []
    for i in range(5):
        w_specs.append(pl.BlockSpec((dims_out[i], dims_in[i]), lambda n: (0, 0)))
        w_specs.append(pl.BlockSpec((dims_out[i],), lambda n: (0,)))
    out = pl.pallas_call(
        _mlp_body,
        grid=(NQ // NB,),
        in_specs=[
            pl.BlockSpec((3, NB), lambda n: (0, n)),
            pl.BlockSpec((3, NB), lambda n: (0, n)),
            pl.BlockSpec((1, NB), lambda n: (0, n)),
            pl.BlockSpec((2 * N_LEVELS, NB), lambda n: (0, n)),
        ] + w_specs,
        out_specs=pl.BlockSpec((257, NB), lambda n: (0, n)),
        out_shape=jax.ShapeDtypeStruct((257, NQ), jnp.float32),
    )(qT, smT, dens, feat, W0, b0, W1, b1, W2, b2, W3, b3, W4, b4)
    return out.T


# speedup vs baseline: 1.3123x; 1.0069x over previous
import jax
import jax.numpy as jnp
import numpy as np
from jax import lax
from jax.experimental import pallas as pl
from jax.experimental.pallas import tpu as pltpu
from jax.experimental.pallas import tpu_sc as plsc

N_LEVELS = 16
LEVEL_DIM = 2
BASE_RES = 16
DESIRED_RES = 2048
LOG2_T = 19
T = 1 << LOG2_T
DIVIDE = 1.5
RADIUS = 4.0 * 0.05
K_NN = 20
PRIME1 = np.uint32(2654435761)
PRIME2 = np.uint32(805459861)

R2 = np.float32(RADIUS * RADIUS)
P1_I32 = np.int32(np.uint32(PRIME1).view(np.int32))
P2_I32 = np.int32(np.uint32(PRIME2).view(np.int32))
_SCALE = 2.0 ** (np.log2(DESIRED_RES / BASE_RES) / (N_LEVELS - 1))
RES_LEVELS = [float(int(np.floor(BASE_RES * _SCALE ** l))) for l in range(N_LEVELS)]

NP_PART = 10000
NQ = 2048
NTILES = 32
QPT = NQ // NTILES
CAP = 64
SUP = 25
NCHUNK = NP_PART // 16
LANES = 16
INF_F32 = np.float32(np.inf)
IMAX = np.int32(2**30)


def _sc_body(px_h, py_h, pz_h, qx_h, qy_h, qz_h, tab,
             smx_o, smy_o, smz_o, dens_o, feat,
             px, py, pz, qx, qy, qz,
             cd2, cidx, csel,
             smxv, smyv, smzv, densv,
             idxb, gbuf, featb, cnt_ref, sem):
    wid = lax.axis_index("s") * 2 + lax.axis_index("c")
    base = wid * QPT

    pltpu.sync_copy(px_h, px)
    pltpu.sync_copy(py_h, py)
    pltpu.sync_copy(pz_h, pz)
    pltpu.sync_copy(qx_h.at[pl.ds(base, QPT)], qx.at[pl.ds(0, QPT)])
    pltpu.sync_copy(qy_h.at[pl.ds(base, QPT)], qy.at[pl.ds(0, QPT)])
    pltpu.sync_copy(qz_h.at[pl.ds(base, QPT)], qz.at[pl.ds(0, QPT)])

    lane = lax.iota(jnp.int32, LANES)
    infv = jnp.full((LANES,), INF_F32, jnp.float32)

    def q_body(qi, _):
        qiv = jnp.full((LANES,), qi, jnp.int32)
        qxv = plsc.load_gather(qx, [qiv])
        qyv = plsc.load_gather(qy, [qiv])
        qzv = plsc.load_gather(qz, [qiv])

        for j in range(CAP // LANES):
            cd2[pl.ds(j * LANES, LANES)] = infv

        cnt_ref[0] = 0

        def chunk(js, _carry):
            o0 = js * (LANES * SUP)
            d2s = []
            for s in range(SUP):
                o = pl.multiple_of(o0 + s * LANES, LANES)
                dx = qxv - px[pl.ds(o, LANES)]
                dy = qyv - py[pl.ds(o, LANES)]
                dz = qzv - pz[pl.ds(o, LANES)]
                d2s.append((dx * dx + dy * dy) + dz * dz)
            mn = d2s[0]
            for s in range(1, SUP):
                mn = jnp.minimum(mn, d2s[s])

            @pl.when(jnp.min(mn) <= R2)
            def _():
                cv = jnp.full((LANES,), cnt_ref[0], jnp.int32)
                for s in range(SUP):
                    m = d2s[s] <= R2
                    ones = jnp.where(m, 1, 0).astype(jnp.int32)
                    pos = cv + plsc.cumsum(ones) - 1
                    pos = jnp.minimum(pos, CAP - 1)
                    plsc.store_scatter(cd2, [pos], d2s[s], mask=m)
                    plsc.store_scatter(cidx, [pos], lane + (o0 + s * LANES), mask=m)
                    cv = cv + plsc.all_reduce_population_count(m)
                cnt_ref[0] = jnp.max(cv)

            return 0

        lax.fori_loop(0, NCHUNK // SUP, chunk, 0)
        cnt_s = cnt_ref[0]

        def big_case(_):
            for j in range(CAP // LANES):
                csel[pl.ds(j * LANES, LANES)] = cd2[pl.ds(j * LANES, LANES)]

            def w_cond(st):
                return st[0] < K_NN

            def w_body(st):
                removed = st[0]
                m0 = csel[pl.ds(0, LANES)]
                for j in range(1, CAP // LANES):
                    m0 = jnp.minimum(m0, csel[pl.ds(j * LANES, LANES)])
                mv = jnp.min(m0)
                mvv = jnp.full((LANES,), mv, jnp.float32)
                ndup = jnp.zeros((LANES,), jnp.int32)
                for j in range(CAP // LANES):
                    cj = csel[pl.ds(j * LANES, LANES)]
                    eq = cj == mvv
                    ndup = ndup + plsc.all_reduce_population_count(eq)
                    csel[pl.ds(j * LANES, LANES)] = jnp.where(eq, infv, cj)
                return removed + jnp.max(ndup), mv

            _, tau = lax.while_loop(w_cond, w_body, (jnp.int32(0), jnp.float32(0)))
            tauv = jnp.full((LANES,), tau, jnp.float32)

            nlt = jnp.zeros((LANES,), jnp.int32)
            for j in range(CAP // LANES):
                cj = cd2[pl.ds(j * LANES, LANES)]
                nlt = nlt + plsc.all_reduce_population_count(cj < tauv)
            need = jnp.int32(K_NN) - jnp.max(nlt)

            def i_cond(st):
                return st[0] < need

            def i_body(st):
                prev = st[1]
                prevv = jnp.full((LANES,), prev, jnp.int32)
                imin = jnp.full((LANES,), IMAX, jnp.int32)
                for j in range(CAP // LANES):
                    cj = cd2[pl.ds(j * LANES, LANES)]
                    ij = cidx[pl.ds(j * LANES, LANES)]
                    cand = jnp.where((cj == tauv) & (ij > prevv), ij, IMAX)
                    imin = jnp.minimum(imin, cand)
                return st[0] + 1, jnp.min(imin)

            _, icut = lax.while_loop(i_cond, i_body, (jnp.int32(0), jnp.int32(-1)))
            return tau, icut

        def small_case(_):
            return jnp.float32(R2), IMAX

        tau, icut = lax.cond(cnt_s > K_NN, big_case, small_case, None)
        tauv = jnp.full((LANES,), tau, jnp.float32)
        icutv = jnp.full((LANES,), icut, jnp.int32)

        ws = jnp.zeros((LANES,), jnp.float32)
        sx = jnp.zeros((LANES,), jnp.float32)
        sy = jnp.zeros((LANES,), jnp.float32)
        sz = jnp.zeros((LANES,), jnp.float32)
        nn = jnp.zeros((LANES,), jnp.int32)
        for j in range(CAP // LANES):
            d2c = cd2[pl.ds(j * LANES, LANES)]
            pic = cidx[pl.ds(j * LANES, LANES)]
            sel = (d2c < tauv) | ((d2c == tauv) & (pic <= icutv))
            live = sel & (d2c != 0.0)
            w = jnp.maximum(R2 - d2c, 0.0)
            wm = jnp.where(live, (w * w) * w, 0.0)
            ws = ws + wm
            nn = nn + jnp.where(live, 1, 0).astype(jnp.int32)
            gx = plsc.load_gather(px, [pic], mask=sel)
            gy = plsc.load_gather(py, [pic], mask=sel)
            gz = plsc.load_gather(pz, [pic], mask=sel)
            sx = sx + wm * gx
            sy = sy + wm * gy
            sz = sz + wm * gz

        wsumv = jnp.full((LANES,), jnp.sum(ws), jnp.float32)
        hasv = jnp.full((LANES,), jnp.sum(nn), jnp.int32) > 0
        denv = jnp.maximum(wsumv, jnp.float32(1e-12))
        resx = jnp.where(hasv, jnp.full((LANES,), jnp.sum(sx), jnp.float32) / denv, qxv)
        resy = jnp.where(hasv, jnp.full((LANES,), jnp.sum(sy), jnp.float32) / denv, qyv)
        resz = jnp.where(hasv, jnp.full((LANES,), jnp.sum(sz), jnp.float32) / denv, qzv)
        lane0 = lane == 0
        plsc.store_scatter(smxv, [qiv], resx, mask=lane0)
        plsc.store_scatter(smyv, [qiv], resy, mask=lane0)
        plsc.store_scatter(smzv, [qiv], resz, mask=lane0)
        plsc.store_scatter(densv, [qiv], wsumv, mask=lane0)
        return 0

    lax.fori_loop(0, QPT, q_body, 0)

    pltpu.sync_copy(smxv.at[pl.ds(0, QPT)], smx_o.at[pl.ds(base, QPT)])
    pltpu.sync_copy(smyv.at[pl.ds(0, QPT)], smy_o.at[pl.ds(base, QPT)])
    pltpu.sync_copy(smzv.at[pl.ds(0, QPT)], smz_o.at[pl.ds(base, QPT)])
    pltpu.sync_copy(densv.at[pl.ds(0, QPT)], dens_o.at[pl.ds(base, QPT)])

    rowvs = [lane + c * LANES for c in range(8)]

    def h_body(ch, _):
        o = pl.multiple_of(ch * LANES, LANES)
        xin = qx[pl.ds(o, LANES)] / DIVIDE
        yin = qy[pl.ds(o, LANES)] / DIVIDE
        zin = qz[pl.ds(o, LANES)] / DIVIDE
        xnx = jnp.clip((xin + 1.0) * 0.5, 0.0, 1.0 - 1e-6)
        xny = jnp.clip((yin + 1.0) * 0.5, 0.0, 1.0 - 1e-6)
        xnz = jnp.clip((zin + 1.0) * 0.5, 0.0, 1.0 - 1e-6)

        def corners(l):
            res = np.float32(RES_LEVELS[l])
            ix = (xnx * res).astype(jnp.int32)
            iy = (xny * res).astype(jnp.int32)
            iz = (xnz * res).astype(jnp.int32)
            return ix, iy, iz

        for l in range(N_LEVELS):
            ix, iy, iz = corners(l)
            a = [ix, ix + 1]
            b = [iy * P1_I32, (iy + 1) * P1_I32]
            c3 = [iz * P2_I32, (iz + 1) * P2_I32]
            for c in range(8):
                h = (a[c & 1] ^ b[(c >> 1) & 1] ^ c3[(c >> 2) & 1]) & jnp.int32(T - 1)
                g2 = ((h >> 7) << 8) + (h & 127) + jnp.int32(l * 2 * T)
                idxb[pl.ds(l * 256 + c * LANES, LANES)] = g2
                idxb[pl.ds(l * 256 + 128 + c * LANES, LANES)] = g2 + 128

        descs = []
        for l in range(N_LEVELS):
            descs.append(pltpu.async_copy(
                tab.at[idxb.at[pl.ds(l * 256, 128)]],
                gbuf.at[pl.ds(l * 256, 128)], sem))
            descs.append(pltpu.async_copy(
                tab.at[idxb.at[pl.ds(l * 256 + 128, 128)]],
                gbuf.at[pl.ds(l * 256 + 128, 128)], sem))
        for d in descs:
            d.wait()

        for l in range(N_LEVELS):
            res = np.float32(RES_LEVELS[l])
            posx = xnx * res
            posy = xny * res
            posz = xnz * res
            fx = posx - posx.astype(jnp.int32).astype(jnp.float32)
            fy = posy - posy.astype(jnp.int32).astype(jnp.float32)
            fz = posz - posz.astype(jnp.int32).astype(jnp.float32)
            wxs = [1.0 - fx, fx]
            wys = [1.0 - fy, fy]
            wzs = [1.0 - fz, fz]
            f0 = jnp.zeros((LANES,), jnp.float32)
            f1 = jnp.zeros((LANES,), jnp.float32)
            for c in range(8):
                wt = wxs[c & 1] * wys[(c >> 1) & 1] * wzs[(c >> 2) & 1]
                rowv = rowvs[c] + l * 256
                g0 = plsc.load_gather(gbuf, [rowv])
                g1 = plsc.load_gather(gbuf, [rowv + 128])
                f0 = f0 + wt * g0
                f1 = f1 + wt * g1
            featb[pl.ds((2 * l) * QPT + o, LANES)] = f0
            featb[pl.ds((2 * l + 1) * QPT + o, LANES)] = f1
        return 0

    lax.fori_loop(0, QPT // LANES, h_body, 0)
    for r in range(2 * N_LEVELS):
        pltpu.sync_copy(featb.at[pl.ds(r * QPT, QPT)],
                        feat.at[pl.ds(r * NQ + base, QPT)])


def _softplus100(x):
    z = 100.0 * x
    return (jnp.maximum(z, 0.0) + jnp.log1p(jnp.exp(-jnp.abs(z)))) / 100.0


def _mlp_body(inp, sm, dn, ft, W0, b0, W1, b1, W2, b2, W3, b3, W4, b4, out):
    hit = inp[...]
    smo = sm[...]
    dnv = dn[...]
    rows = [hit]
    for i in range(10):
        f = float(2.0 ** i)
        rows.append(jnp.sin(f * hit))
        rows.append(jnp.cos(f * hit))
    rows.append(smo)
    for i in range(10):
        f = float(2.0 ** i)
        rows.append(jnp.sin(f * smo))
        rows.append(jnp.cos(f * smo))
    rows.append(dnv)
    for i in range(4):
        f = float(2.0 ** i)
        rows.append(jnp.sin(f * dnv))
        rows.append(jnp.cos(f * dnv))
    rows.append(ft[...])
    x = jnp.concatenate(rows, axis=0)
    Ws = [W0, W1, W2, W3, W4]
    bs = [b0, b1, b2, b3, b4]
    for i in range(5):
        x = jnp.dot(Ws[i][...], x, preferred_element_type=jnp.float32)
        x = x + bs[i][...][:, None]
        if i < 4:
            x = _softplus100(x)
    out[...] = x


def kernel(input, physical_particles, tables, W0, b0, W1, b1, W2, b2, W3, b3, W4, b4):
    B, N, _ = input.shape
    inp2 = input.reshape(B * N, 3)
    qT = inp2.T
    pT = physical_particles.T
    tab = tables.reshape(N_LEVELS, T // 128, 128, LEVEL_DIM)
    tab = tab.transpose(0, 1, 3, 2).reshape(N_LEVELS * T * LEVEL_DIM)
    px_h, py_h, pz_h = pT[0], pT[1], pT[2]
    qx_h, qy_h, qz_h = qT[0], qT[1], qT[2]

    sc = pl.kernel(
        _sc_body,
        out_type=[
            jax.ShapeDtypeStruct((NQ,), jnp.float32),
            jax.ShapeDtypeStruct((NQ,), jnp.float32),
            jax.ShapeDtypeStruct((NQ,), jnp.float32),
            jax.ShapeDtypeStruct((NQ,), jnp.float32),
            jax.ShapeDtypeStruct((2 * N_LEVELS * NQ,), jnp.float32),
        ],
        mesh=plsc.VectorSubcoreMesh(
            core_axis_name="c", subcore_axis_name="s", num_cores=2, num_subcores=16
        ),
        compiler_params=pltpu.CompilerParams(needs_layout_passes=False, use_tc_tiling_on_sc=False),
        scratch_types=[
            pltpu.VMEM((NP_PART,), jnp.float32),
            pltpu.VMEM((NP_PART,), jnp.float32),
            pltpu.VMEM((NP_PART,), jnp.float32),
            pltpu.VMEM((128,), jnp.float32),
            pltpu.VMEM((128,), jnp.float32),
            pltpu.VMEM((128,), jnp.float32),
            pltpu.VMEM((CAP,), jnp.float32),
            pltpu.VMEM((CAP,), jnp.int32),
            pltpu.VMEM((CAP,), jnp.float32),
            pltpu.VMEM((128,), jnp.float32),
            pltpu.VMEM((128,), jnp.float32),
            pltpu.VMEM((128,), jnp.float32),
            pltpu.VMEM((128,), jnp.float32),
            pltpu.VMEM((N_LEVELS * 256,), jnp.int32),
            pltpu.VMEM((N_LEVELS * 256,), jnp.float32),
            pltpu.VMEM((2 * N_LEVELS * QPT,), jnp.float32),
            pltpu.SMEM((8,), jnp.int32),
            pltpu.SemaphoreType.DMA,
        ],
    )
    smx, smy, smz, densf, featf = sc(px_h, py_h, pz_h, qx_h, qy_h, qz_h, tab)
    smT = jnp.stack([smx, smy, smz])
    dens = densf.reshape(1, NQ)
    feat = featf.reshape(2 * N_LEVELS, NQ)

    NB = 512
    dims_in = [167, 256, 256, 256, 256]
    dims_out = [256, 256, 256, 256, 257]
    w_specs = []
    for i in range(5):
        w_specs.append(pl.BlockSpec((dims_out[i], dims_in[i]), lambda n: (0, 0)))
        w_specs.append(pl.BlockSpec((dims_out[i],), lambda n: (0,)))
    out = pl.pallas_call(
        _mlp_body,
        grid=(NQ // NB,),
        in_specs=[
            pl.BlockSpec((3, NB), lambda n: (0, n)),
            pl.BlockSpec((3, NB), lambda n: (0, n)),
            pl.BlockSpec((1, NB), lambda n: (0, n)),
            pl.BlockSpec((2 * N_LEVELS, NB), lambda n: (0, n)),
        ] + w_specs,
        out_specs=pl.BlockSpec((257, NB), lambda n: (0, n)),
        out_shape=jax.ShapeDtypeStruct((257, NQ), jnp.float32),
    )(qT, smT, dens, feat, W0, b0, W1, b1, W2, b2, W3, b3, W4, b4)
    return out.T


# speedup vs baseline: 1.3164x; 1.0031x over previous
import jax
import jax.numpy as jnp
import numpy as np
from jax import lax
from jax.experimental import pallas as pl
from jax.experimental.pallas import tpu as pltpu
from jax.experimental.pallas import tpu_sc as plsc

N_LEVELS = 16
LEVEL_DIM = 2
BASE_RES = 16
DESIRED_RES = 2048
LOG2_T = 19
T = 1 << LOG2_T
DIVIDE = 1.5
RADIUS = 4.0 * 0.05
K_NN = 20
PRIME1 = np.uint32(2654435761)
PRIME2 = np.uint32(805459861)

R2 = np.float32(RADIUS * RADIUS)
P1_I32 = np.int32(np.uint32(PRIME1).view(np.int32))
P2_I32 = np.int32(np.uint32(PRIME2).view(np.int32))
_SCALE = 2.0 ** (np.log2(DESIRED_RES / BASE_RES) / (N_LEVELS - 1))
RES_LEVELS = [float(int(np.floor(BASE_RES * _SCALE ** l))) for l in range(N_LEVELS)]

NP_PART = 10000
NQ = 2048
NTILES = 32
QPT = NQ // NTILES
CAP = 64
SUP = 25
NCHUNK = NP_PART // 16
LANES = 16
INF_F32 = np.float32(np.inf)
IMAX = np.int32(2**30)


def _sc_body(px_h, py_h, pz_h, qx_h, qy_h, qz_h, tab,
             smx_o, smy_o, smz_o, dens_o, feat,
             px, py, pz, qx, qy, qz,
             cd2, cidx, csel,
             smxv, smyv, smzv, densv,
             idxb, gbuf, featb, cnt_ref, sem):
    wid = lax.axis_index("s") * 2 + lax.axis_index("c")
    base = wid * QPT

    pltpu.sync_copy(px_h, px)
    pltpu.sync_copy(py_h, py)
    pltpu.sync_copy(pz_h, pz)
    pltpu.sync_copy(qx_h.at[pl.ds(base, QPT)], qx.at[pl.ds(0, QPT)])
    pltpu.sync_copy(qy_h.at[pl.ds(base, QPT)], qy.at[pl.ds(0, QPT)])
    pltpu.sync_copy(qz_h.at[pl.ds(base, QPT)], qz.at[pl.ds(0, QPT)])

    lane = lax.iota(jnp.int32, LANES)
    infv = jnp.full((LANES,), INF_F32, jnp.float32)

    def q_body(qi, _):
        qiv = jnp.full((LANES,), qi, jnp.int32)
        qxv = plsc.load_gather(qx, [qiv])
        qyv = plsc.load_gather(qy, [qiv])
        qzv = plsc.load_gather(qz, [qiv])

        for j in range(CAP // LANES):
            cd2[pl.ds(j * LANES, LANES)] = infv

        cnt_ref[0] = 0

        def chunk(js, _carry):
            o0 = js * (LANES * SUP)
            d2s = []
            for s in range(SUP):
                o = pl.multiple_of(o0 + s * LANES, LANES)
                dx = qxv - px[pl.ds(o, LANES)]
                dy = qyv - py[pl.ds(o, LANES)]
                dz = qzv - pz[pl.ds(o, LANES)]
                d2s.append((dx * dx + dy * dy) + dz * dz)
            mn = d2s[0]
            for s in range(1, SUP):
                mn = jnp.minimum(mn, d2s[s])

            @pl.when(jnp.min(mn) <= R2)
            def _():
                cv = jnp.full((LANES,), cnt_ref[0], jnp.int32)
                for s in range(SUP):
                    m = d2s[s] <= R2
                    ones = jnp.where(m, 1, 0).astype(jnp.int32)
                    pos = cv + plsc.cumsum(ones) - 1
                    pos = jnp.minimum(pos, CAP - 1)
                    plsc.store_scatter(cd2, [pos], d2s[s], mask=m)
                    plsc.store_scatter(cidx, [pos], lane + (o0 + s * LANES), mask=m)
                    cv = cv + plsc.all_reduce_population_count(m)
                cnt_ref[0] = jnp.max(cv)

            return 0

        lax.fori_loop(0, NCHUNK // SUP, chunk, 0)
        cnt_s = cnt_ref[0]

        def big_case(_):
            for j in range(CAP // LANES):
                csel[pl.ds(j * LANES, LANES)] = cd2[pl.ds(j * LANES, LANES)]

            def w_cond(st):
                return st[0] < K_NN

            def w_body(st):
                removed = st[0]
                m0 = csel[pl.ds(0, LANES)]
                for j in range(1, CAP // LANES):
                    m0 = jnp.minimum(m0, csel[pl.ds(j * LANES, LANES)])
                mv = jnp.min(m0)
                mvv = jnp.full((LANES,), mv, jnp.float32)
                ndup = jnp.zeros((LANES,), jnp.int32)
                for j in range(CAP // LANES):
                    cj = csel[pl.ds(j * LANES, LANES)]
                    eq = cj == mvv
                    ndup = ndup + plsc.all_reduce_population_count(eq)
                    csel[pl.ds(j * LANES, LANES)] = jnp.where(eq, infv, cj)
                return removed + jnp.max(ndup), mv

            _, tau = lax.while_loop(w_cond, w_body, (jnp.int32(0), jnp.float32(0)))
            tauv = jnp.full((LANES,), tau, jnp.float32)

            nlt = jnp.zeros((LANES,), jnp.int32)
            for j in range(CAP // LANES):
                cj = cd2[pl.ds(j * LANES, LANES)]
                nlt = nlt + plsc.all_reduce_population_count(cj < tauv)
            need = jnp.int32(K_NN) - jnp.max(nlt)

            def i_cond(st):
                return st[0] < need

            def i_body(st):
                prev = st[1]
                prevv = jnp.full((LANES,), prev, jnp.int32)
                imin = jnp.full((LANES,), IMAX, jnp.int32)
                for j in range(CAP // LANES):
                    cj = cd2[pl.ds(j * LANES, LANES)]
                    ij = cidx[pl.ds(j * LANES, LANES)]
                    cand = jnp.where((cj == tauv) & (ij > prevv), ij, IMAX)
                    imin = jnp.minimum(imin, cand)
                return st[0] + 1, jnp.min(imin)

            _, icut = lax.while_loop(i_cond, i_body, (jnp.int32(0), jnp.int32(-1)))
            return tau, icut

        def small_case(_):
            return jnp.float32(R2), IMAX

        tau, icut = lax.cond(cnt_s > K_NN, big_case, small_case, None)
        tauv = jnp.full((LANES,), tau, jnp.float32)
        icutv = jnp.full((LANES,), icut, jnp.int32)

        ws = jnp.zeros((LANES,), jnp.float32)
        sx = jnp.zeros((LANES,), jnp.float32)
        sy = jnp.zeros((LANES,), jnp.float32)
        sz = jnp.zeros((LANES,), jnp.float32)
        nn = jnp.zeros((LANES,), jnp.int32)
        for j in range(CAP // LANES):
            d2c = cd2[pl.ds(j * LANES, LANES)]
            pic = cidx[pl.ds(j * LANES, LANES)]
            sel = (d2c < tauv) | ((d2c == tauv) & (pic <= icutv))
            live = sel & (d2c != 0.0)
            w = jnp.maximum(R2 - d2c, 0.0)
            wm = jnp.where(live, (w * w) * w, 0.0)
            ws = ws + wm
            nn = nn + jnp.where(live, 1, 0).astype(jnp.int32)
            gx = plsc.load_gather(px, [pic], mask=sel)
            gy = plsc.load_gather(py, [pic], mask=sel)
            gz = plsc.load_gather(pz, [pic], mask=sel)
            sx = sx + wm * gx
            sy = sy + wm * gy
            sz = sz + wm * gz

        wsumv = jnp.full((LANES,), jnp.sum(ws), jnp.float32)
        hasv = jnp.full((LANES,), jnp.sum(nn), jnp.int32) > 0
        denv = jnp.maximum(wsumv, jnp.float32(1e-12))
        resx = jnp.where(hasv, jnp.full((LANES,), jnp.sum(sx), jnp.float32) / denv, qxv)
        resy = jnp.where(hasv, jnp.full((LANES,), jnp.sum(sy), jnp.float32) / denv, qyv)
        resz = jnp.where(hasv, jnp.full((LANES,), jnp.sum(sz), jnp.float32) / denv, qzv)
        lane0 = lane == 0
        plsc.store_scatter(smxv, [qiv], resx, mask=lane0)
        plsc.store_scatter(smyv, [qiv], resy, mask=lane0)
        plsc.store_scatter(smzv, [qiv], resz, mask=lane0)
        plsc.store_scatter(densv, [qiv], wsumv, mask=lane0)
        return 0

    lax.fori_loop(0, QPT, q_body, 0)

    pltpu.sync_copy(smxv.at[pl.ds(0, QPT)], smx_o.at[pl.ds(base, QPT)])
    pltpu.sync_copy(smyv.at[pl.ds(0, QPT)], smy_o.at[pl.ds(base, QPT)])
    pltpu.sync_copy(smzv.at[pl.ds(0, QPT)], smz_o.at[pl.ds(base, QPT)])
    pltpu.sync_copy(densv.at[pl.ds(0, QPT)], dens_o.at[pl.ds(base, QPT)])

    rowvs = [lane + c * LANES for c in range(8)]

    def h_body(ch, _):
        o = pl.multiple_of(ch * LANES, LANES)
        xin = qx[pl.ds(o, LANES)] / DIVIDE
        yin = qy[pl.ds(o, LANES)] / DIVIDE
        zin = qz[pl.ds(o, LANES)] / DIVIDE
        xnx = jnp.clip((xin + 1.0) * 0.5, 0.0, 1.0 - 1e-6)
        xny = jnp.clip((yin + 1.0) * 0.5, 0.0, 1.0 - 1e-6)
        xnz = jnp.clip((zin + 1.0) * 0.5, 0.0, 1.0 - 1e-6)

        def corners(l):
            res = np.float32(RES_LEVELS[l])
            ix = (xnx * res).astype(jnp.int32)
            iy = (xny * res).astype(jnp.int32)
            iz = (xnz * res).astype(jnp.int32)
            return ix, iy, iz

        for l in range(N_LEVELS):
            ix, iy, iz = corners(l)
            a = [ix, ix + 1]
            b = [iy * P1_I32, (iy + 1) * P1_I32]
            c3 = [iz * P2_I32, (iz + 1) * P2_I32]
            for c in range(8):
                h = (a[c & 1] ^ b[(c >> 1) & 1] ^ c3[(c >> 2) & 1]) & jnp.int32(T - 1)
                g2 = ((h >> 7) << 8) + (h & 127) + jnp.int32(l * 2 * T)
                idxb[pl.ds(l * 256 + c * LANES, LANES)] = g2
                idxb[pl.ds(l * 256 + 128 + c * LANES, LANES)] = g2 + 128

        descs = []
        for l in range(N_LEVELS):
            descs.append(pltpu.async_copy(
                tab.at[idxb.at[pl.ds(l * 256, 256)]],
                gbuf.at[pl.ds(l * 256, 256)], sem))
        for d in descs:
            d.wait()

        for l in range(N_LEVELS):
            res = np.float32(RES_LEVELS[l])
            posx = xnx * res
            posy = xny * res
            posz = xnz * res
            fx = posx - posx.astype(jnp.int32).astype(jnp.float32)
            fy = posy - posy.astype(jnp.int32).astype(jnp.float32)
            fz = posz - posz.astype(jnp.int32).astype(jnp.float32)
            wxs = [1.0 - fx, fx]
            wys = [1.0 - fy, fy]
            wzs = [1.0 - fz, fz]
            f0 = jnp.zeros((LANES,), jnp.float32)
            f1 = jnp.zeros((LANES,), jnp.float32)
            for c in range(8):
                wt = wxs[c & 1] * wys[(c >> 1) & 1] * wzs[(c >> 2) & 1]
                rowv = rowvs[c] + l * 256
                g0 = plsc.load_gather(gbuf, [rowv])
                g1 = plsc.load_gather(gbuf, [rowv + 128])
                f0 = f0 + wt * g0
                f1 = f1 + wt * g1
            featb[pl.ds((2 * l) * QPT + o, LANES)] = f0
            featb[pl.ds((2 * l + 1) * QPT + o, LANES)] = f1
        return 0

    lax.fori_loop(0, QPT // LANES, h_body, 0)
    for r in range(2 * N_LEVELS):
        pltpu.sync_copy(featb.at[pl.ds(r * QPT, QPT)],
                        feat.at[pl.ds(r * NQ + base, QPT)])


def _softplus100(x):
    z = 100.0 * x
    return (jnp.maximum(z, 0.0) + jnp.log1p(jnp.exp(-jnp.abs(z)))) / 100.0


def _mlp_body(inp, sm, dn, ft, W0, b0, W1, b1, W2, b2, W3, b3, W4, b4, out):
    hit = inp[...]
    smo = sm[...]
    dnv = dn[...]
    rows = [hit]
    for i in range(10):
        f = float(2.0 ** i)
        rows.append(jnp.sin(f * hit))
        rows.append(jnp.cos(f * hit))
    rows.append(smo)
    for i in range(10):
        f = float(2.0 ** i)
        rows.append(jnp.sin(f * smo))
        rows.append(jnp.cos(f * smo))
    rows.append(dnv)
    for i in range(4):
        f = float(2.0 ** i)
        rows.append(jnp.sin(f * dnv))
        rows.append(jnp.cos(f * dnv))
    rows.append(ft[...])
    x = jnp.concatenate(rows, axis=0)
    Ws = [W0, W1, W2, W3, W4]
    bs = [b0, b1, b2, b3, b4]
    for i in range(5):
        x = jnp.dot(Ws[i][...], x, preferred_element_type=jnp.float32)
        x = x + bs[i][...][:, None]
        if i < 4:
            x = _softplus100(x)
    out[...] = x


def kernel(input, physical_particles, tables, W0, b0, W1, b1, W2, b2, W3, b3, W4, b4):
    B, N, _ = input.shape
    inp2 = input.reshape(B * N, 3)
    qT = inp2.T
    pT = physical_particles.T
    tab = tables.reshape(N_LEVELS, T // 128, 128, LEVEL_DIM)
    tab = tab.transpose(0, 1, 3, 2).reshape(N_LEVELS * T * LEVEL_DIM)
    px_h, py_h, pz_h = pT[0], pT[1], pT[2]
    qx_h, qy_h, qz_h = qT[0], qT[1], qT[2]

    sc = pl.kernel(
        _sc_body,
        out_type=[
            jax.ShapeDtypeStruct((NQ,), jnp.float32),
            jax.ShapeDtypeStruct((NQ,), jnp.float32),
            jax.ShapeDtypeStruct((NQ,), jnp.float32),
            jax.ShapeDtypeStruct((NQ,), jnp.float32),
            jax.ShapeDtypeStruct((2 * N_LEVELS * NQ,), jnp.float32),
        ],
        mesh=plsc.VectorSubcoreMesh(
            core_axis_name="c", subcore_axis_name="s", num_cores=2, num_subcores=16
        ),
        compiler_params=pltpu.CompilerParams(needs_layout_passes=False, use_tc_tiling_on_sc=False),
        scratch_types=[
            pltpu.VMEM((NP_PART,), jnp.float32),
            pltpu.VMEM((NP_PART,), jnp.float32),
            pltpu.VMEM((NP_PART,), jnp.float32),
            pltpu.VMEM((128,), jnp.float32),
            pltpu.VMEM((128,), jnp.float32),
            pltpu.VMEM((128,), jnp.float32),
            pltpu.VMEM((CAP,), jnp.float32),
            pltpu.VMEM((CAP,), jnp.int32),
            pltpu.VMEM((CAP,), jnp.float32),
            pltpu.VMEM((128,), jnp.float32),
            pltpu.VMEM((128,), jnp.float32),
            pltpu.VMEM((128,), jnp.float32),
            pltpu.VMEM((128,), jnp.float32),
            pltpu.VMEM((N_LEVELS * 256,), jnp.int32),
            pltpu.VMEM((N_LEVELS * 256,), jnp.float32),
            pltpu.VMEM((2 * N_LEVELS * QPT,), jnp.float32),
            pltpu.SMEM((8,), jnp.int32),
            pltpu.SemaphoreType.DMA,
        ],
    )
    smx, smy, smz, densf, featf = sc(px_h, py_h, pz_h, qx_h, qy_h, qz_h, tab)
    smT = jnp.stack([smx, smy, smz])
    dens = densf.reshape(1, NQ)
    feat = featf.reshape(2 * N_LEVELS, NQ)

    NB = 512
    dims_in = [167, 256, 256, 256, 256]
    dims_out = [256, 256, 256, 256, 257]
    w_specs = []
    for i in range(5):
        w_specs.append(pl.BlockSpec((dims_out[i], dims_in[i]), lambda n: (0, 0)))
        w_specs.append(pl.BlockSpec((dims_out[i],), lambda n: (0,)))
    out = pl.pallas_call(
        _mlp_body,
        grid=(NQ // NB,),
        in_specs=[
            pl.BlockSpec((3, NB), lambda n: (0, n)),
            pl.BlockSpec((3, NB), lambda n: (0, n)),
            pl.BlockSpec((1, NB), lambda n: (0, n)),
            pl.BlockSpec((2 * N_LEVELS, NB), lambda n: (0, n)),
        ] + w_specs,
        out_specs=pl.BlockSpec((257, NB), lambda n: (0, n)),
        out_shape=jax.ShapeDtypeStruct((257, NQ), jnp.float32),
    )(qT, smT, dens, feat, W0, b0, W1, b1, W2, b2, W3, b3, W4, b4)
    return out.T
